# Initial kernel scaffold; baseline (speedup 1.0000x reference)
import jax
import jax.numpy as jnp
from jax.experimental import pallas as pl


def kernel(hidden_states, pos_table):
    raise NotImplementedError("write your pallas kernel here")


# speedup vs baseline: 1.5229x; 1.5229x over previous
import functools

import jax
import jax.numpy as jnp
from jax import lax
from jax.experimental import pallas as pl
from jax.experimental.pallas import tpu as pltpu
from jax.experimental.pallas import tpu_sc as plsc


@functools.lru_cache(maxsize=None)
def _make_flip_gather(L: int, H: int):
    info = plsc.get_sparse_core_info()
    NC, NS, LANES = info.num_cores, info.num_subcores, info.num_lanes
    NW = NC * NS
    rows_per_w = L // NW
    C = 64
    n_chunks = rows_per_w // C

    mesh = plsc.VectorSubcoreMesh(core_axis_name="c", subcore_axis_name="s")

    @functools.partial(
        pl.kernel,
        mesh=mesh,
        out_type=jax.ShapeDtypeStruct((L, H), jnp.float32),
        scratch_types=[
            pltpu.VMEM((C,), jnp.int32),
            pltpu.VMEM((C, H), jnp.float32),
            pltpu.SemaphoreType.DMA,
        ],
    )
    def flip_gather(table_hbm, out_hbm, idx_v, rows_v, sem):
        wid = lax.axis_index("s") * NC + lax.axis_index("c")
        base = wid * rows_per_w
        for c in range(n_chunks):
            out_base = base + c * C
            top = (L - 1) - out_base
            for i in range(C // LANES):
                idx_v[pl.ds(i * LANES, LANES)] = (
                    top - i * LANES - lax.iota(jnp.int32, LANES)
                )
            pltpu.async_copy(table_hbm.at[idx_v], rows_v, sem).wait()
            pltpu.sync_copy(rows_v, out_hbm.at[pl.ds(out_base, C)])

    return flip_gather


def kernel(hidden_states, pos_table):
    L = hidden_states.shape[1]
    H = pos_table.shape[1]
    out = _make_flip_gather(L, H)(pos_table)
    return out.reshape(1, L, H)
